# Initial kernel scaffold; baseline (speedup 1.0000x reference)
import jax
import jax.numpy as jnp
from jax.experimental import pallas as pl


def kernel(x, W0, W1, Wc):
    raise NotImplementedError("write your pallas kernel here")


# speedup vs baseline: 8.9174x; 8.9174x over previous
import jax
import jax.numpy as jnp
from jax import lax
from jax.experimental import pallas as pl

_B, _N, _K, _H = 8, 1024, 20, 64
_NT = 4
_TN = _N // _NT
_EPS = 1e-5
_CNT0 = float(_B * _N * _K)
_CNT2 = float(_B * _N)


def _leaky(v):
    return jnp.where(v >= 0, v, 0.2 * v)


def _knn_kernel(x_ref, xt_ref, w0at_ref, wbat_ref, xj_ref, s0_ref, ss0_ref):
    b = pl.program_id(0)
    xr = x_ref[0]
    xi = xt_ref[0]
    G = lax.dot_general(xi, xi, (((1,), (1,)), ((), ())),
                        preferred_element_type=jnp.float32)
    xx_row = jnp.sum(xr * xr, axis=0, keepdims=True)
    score = 2.0 * G - xx_row
    colid = lax.broadcasted_iota(jnp.int32, (_N, _N), 1)
    bterm = jnp.dot(xi, wbat_ref[...], preferred_element_type=jnp.float32)
    s0 = jnp.zeros((1, _H), jnp.float32)
    ss0 = jnp.zeros((1, _H), jnp.float32)
    for t in range(_K):
        m = jnp.max(score, axis=1, keepdims=True)
        eq = score == m
        j = jnp.min(jnp.where(eq, colid, _N), axis=1, keepdims=True)
        oh = colid == j
        gx = jnp.dot(oh.astype(jnp.float32), xi,
                     preferred_element_type=jnp.float32)
        xj_ref[0, t] = gx
        y0t = jnp.dot(gx, w0at_ref[...],
                      preferred_element_type=jnp.float32) + bterm
        s0 = s0 + jnp.sum(y0t, axis=0, keepdims=True)
        ss0 = ss0 + jnp.sum(y0t * y0t, axis=0, keepdims=True)
        score = jnp.where(oh, -jnp.inf, score)

    @pl.when(b == 0)
    def _init():
        s0_ref[...] = jnp.zeros_like(s0_ref)
        ss0_ref[...] = jnp.zeros_like(ss0_ref)

    s0_ref[...] += s0
    ss0_ref[...] += ss0


def _feat_kernel(xt_ref, xj_ref, s0_ref, ss0_ref, w0at_ref, wbat_ref,
                 w1ct_ref, x1_ref, s1_ref, ss1_ref):
    b = pl.program_id(0)
    i = pl.program_id(1)
    m0 = s0_ref[...] / _CNT0
    v0 = ss0_ref[...] / _CNT0 - m0 * m0
    r0 = 1.0 / jnp.sqrt(v0 + _EPS)
    xi = xt_ref[0]
    xj = xj_ref[0]
    bterm = jnp.dot(xi, wbat_ref[...], preferred_element_type=jnp.float32)
    y0 = (jnp.dot(xj.reshape(_K * _TN, 8), w0at_ref[...],
                  preferred_element_type=jnp.float32)
          .reshape(_K, _TN, _H) + bterm[None])
    y0n = _leaky((y0 - m0) * r0)
    y0f = y0n.reshape(_K * _TN, _H)
    acc = jnp.zeros((_K, _TN, _H), jnp.float32)
    for c in range(6):
        contrib = (jnp.dot(y0f, w1ct_ref[c],
                           preferred_element_type=jnp.float32)
                   .reshape(_K, _TN, _H))
        if c < 3:
            pc = xj[:, :, c:c + 1] - xi[None, :, c:c + 1]
        else:
            pc = jnp.broadcast_to(xi[None, :, c - 3:c - 2], (_K, _TN, 1))
        acc = acc + contrib * pc
    x1_ref[0] = jnp.max(acc, axis=0)

    @pl.when(jnp.logical_and(b == 0, i == 0))
    def _init():
        s1_ref[...] = jnp.zeros_like(s1_ref)
        ss1_ref[...] = jnp.zeros_like(ss1_ref)

    accf = acc.reshape(_K * _TN, _H)
    s1_ref[...] += jnp.sum(accf, axis=0, keepdims=True)
    ss1_ref[...] += jnp.sum(accf * accf, axis=0, keepdims=True)


def _head_kernel(x1_ref, s1_ref, ss1_ref, wct_ref, out_ref):
    m1 = s1_ref[...] / _CNT0
    v1 = ss1_ref[...] / _CNT0 - m1 * m1
    r1 = 1.0 / jnp.sqrt(v1 + _EPS)
    x1 = x1_ref[...].reshape(_B * _N, _H)
    x1n = _leaky((x1 - m1) * r1)
    tt = jnp.dot(x1n, wct_ref[...], preferred_element_type=jnp.float32)
    m2 = jnp.sum(tt, axis=0, keepdims=True) / _CNT2
    v2 = jnp.sum(tt * tt, axis=0, keepdims=True) / _CNT2 - m2 * m2
    out = _leaky((tt - m2) * (1.0 / jnp.sqrt(v2 + _EPS)))
    out_ref[...] = out.reshape(_B, _N, 8)


def kernel(x, W0, W1, Wc):
    xt = jnp.pad(jnp.transpose(x, (0, 2, 1)), ((0, 0), (0, 0), (0, 5)))
    W0a = W0[:, :3]
    W0b = W0[:, 3:]
    w0at = jnp.pad(W0a.T, ((0, 5), (0, 0)))
    wbat = jnp.pad((W0b - W0a).T, ((0, 5), (0, 0)))
    w1ct = jnp.transpose(W1.reshape(_H, 6, _H), (1, 2, 0))
    wct = jnp.pad(Wc.T, ((0, 0), (0, 5)))

    xj, s0, ss0 = pl.pallas_call(
        _knn_kernel,
        grid=(_B,),
        in_specs=[pl.BlockSpec((1, 3, _N), lambda b: (b, 0, 0)),
                  pl.BlockSpec((1, _N, 8), lambda b: (b, 0, 0)),
                  pl.BlockSpec((8, _H), lambda b: (0, 0)),
                  pl.BlockSpec((8, _H), lambda b: (0, 0))],
        out_specs=[pl.BlockSpec((1, _K, _N, 8), lambda b: (b, 0, 0, 0)),
                   pl.BlockSpec((1, _H), lambda b: (0, 0)),
                   pl.BlockSpec((1, _H), lambda b: (0, 0))],
        out_shape=[jax.ShapeDtypeStruct((_B, _K, _N, 8), jnp.float32),
                   jax.ShapeDtypeStruct((1, _H), jnp.float32),
                   jax.ShapeDtypeStruct((1, _H), jnp.float32)],
    )(x, xt, w0at, wbat)

    x1, s1, ss1 = pl.pallas_call(
        _feat_kernel,
        grid=(_B, _NT),
        in_specs=[pl.BlockSpec((1, _TN, 8), lambda b, i: (b, i, 0)),
                  pl.BlockSpec((1, _K, _TN, 8), lambda b, i: (b, 0, i, 0)),
                  pl.BlockSpec((1, _H), lambda b, i: (0, 0)),
                  pl.BlockSpec((1, _H), lambda b, i: (0, 0)),
                  pl.BlockSpec((8, _H), lambda b, i: (0, 0)),
                  pl.BlockSpec((8, _H), lambda b, i: (0, 0)),
                  pl.BlockSpec((6, _H, _H), lambda b, i: (0, 0, 0))],
        out_specs=[pl.BlockSpec((1, _TN, _H), lambda b, i: (b, i, 0)),
                   pl.BlockSpec((1, _H), lambda b, i: (0, 0)),
                   pl.BlockSpec((1, _H), lambda b, i: (0, 0))],
        out_shape=[jax.ShapeDtypeStruct((_B, _N, _H), jnp.float32),
                   jax.ShapeDtypeStruct((1, _H), jnp.float32),
                   jax.ShapeDtypeStruct((1, _H), jnp.float32)],
    )(xt, xj, s0, ss0, w0at, wbat, w1ct)

    res = pl.pallas_call(
        _head_kernel,
        out_shape=jax.ShapeDtypeStruct((_B, _N, 8), jnp.float32),
    )(x1, s1, ss1, wct)
    return jnp.transpose(res[:, :, :3], (0, 2, 1))


# speedup vs baseline: 9.3766x; 1.0515x over previous
import jax
import jax.numpy as jnp
from jax import lax
from jax.experimental import pallas as pl

_B, _N, _K, _H = 8, 1024, 20, 64
_NT = 4
_TN = _N // _NT
_EPS = 1e-5
_CNT0 = float(_B * _N * _K)
_CNT2 = float(_B * _N)


def _leaky(v):
    return jnp.where(v >= 0, v, 0.2 * v)


def _knn_kernel(x_ref, xt_ref, w0at_ref, wbat_ref, xj_ref, s0_ref, ss0_ref):
    b = pl.program_id(0)
    xr = x_ref[0]
    xi = xt_ref[0]
    G = lax.dot_general(xi, xi, (((1,), (1,)), ((), ())),
                        preferred_element_type=jnp.float32)
    xx_row = jnp.sum(xr * xr, axis=0, keepdims=True)
    xx_col = jnp.transpose(xx_row)
    inner = -2.0 * G
    score = (-xx_col - inner) - xx_row
    colid = lax.broadcasted_iota(jnp.int32, (_N, _N), 1)
    bterm = jnp.dot(xi, wbat_ref[...], preferred_element_type=jnp.float32)
    s0 = jnp.zeros((1, _H), jnp.float32)
    ss0 = jnp.zeros((1, _H), jnp.float32)
    for t in range(_K):
        j = jnp.argmax(score, axis=1, keepdims=True)
        oh = colid == j
        gx = jnp.dot(oh.astype(jnp.float32), xi,
                     preferred_element_type=jnp.float32)
        xj_ref[0, t] = gx
        y0t = jnp.dot(gx, w0at_ref[...],
                      preferred_element_type=jnp.float32) + bterm
        s0 = s0 + jnp.sum(y0t, axis=0, keepdims=True)
        ss0 = ss0 + jnp.sum(y0t * y0t, axis=0, keepdims=True)
        score = jnp.where(oh, -jnp.inf, score)

    @pl.when(b == 0)
    def _init():
        s0_ref[...] = jnp.zeros_like(s0_ref)
        ss0_ref[...] = jnp.zeros_like(ss0_ref)

    s0_ref[...] += s0
    ss0_ref[...] += ss0


def _feat_kernel(xt_ref, xj_ref, s0_ref, ss0_ref, w0at_ref, wbat_ref,
                 w1ct_ref, x1_ref, s1_ref, ss1_ref):
    b = pl.program_id(0)
    i = pl.program_id(1)
    m0 = s0_ref[...] / _CNT0
    v0 = ss0_ref[...] / _CNT0 - m0 * m0
    r0 = 1.0 / jnp.sqrt(v0 + _EPS)
    xi = xt_ref[0]
    xj = xj_ref[0]
    bterm = jnp.dot(xi, wbat_ref[...], preferred_element_type=jnp.float32)
    y0 = (jnp.dot(xj.reshape(_K * _TN, 8), w0at_ref[...],
                  preferred_element_type=jnp.float32)
          .reshape(_K, _TN, _H) + bterm[None])
    y0n = _leaky((y0 - m0) * r0)
    y0f = y0n.reshape(_K * _TN, _H)
    acc = jnp.zeros((_K, _TN, _H), jnp.float32)
    for c in range(6):
        contrib = (jnp.dot(y0f, w1ct_ref[c],
                           preferred_element_type=jnp.float32)
                   .reshape(_K, _TN, _H))
        if c < 3:
            pc = xj[:, :, c:c + 1] - xi[None, :, c:c + 1]
        else:
            pc = jnp.broadcast_to(xi[None, :, c - 3:c - 2], (_K, _TN, 1))
        acc = acc + contrib * pc
    x1_ref[0] = jnp.max(acc, axis=0)

    @pl.when(jnp.logical_and(b == 0, i == 0))
    def _init():
        s1_ref[...] = jnp.zeros_like(s1_ref)
        ss1_ref[...] = jnp.zeros_like(ss1_ref)

    accf = acc.reshape(_K * _TN, _H)
    s1_ref[...] += jnp.sum(accf, axis=0, keepdims=True)
    ss1_ref[...] += jnp.sum(accf * accf, axis=0, keepdims=True)


def _head_kernel(x1_ref, s1_ref, ss1_ref, wct_ref, out_ref):
    m1 = s1_ref[...] / _CNT0
    v1 = ss1_ref[...] / _CNT0 - m1 * m1
    r1 = 1.0 / jnp.sqrt(v1 + _EPS)
    x1 = x1_ref[...].reshape(_B * _N, _H)
    x1n = _leaky((x1 - m1) * r1)
    tt = jnp.dot(x1n, wct_ref[...], preferred_element_type=jnp.float32)
    m2 = jnp.sum(tt, axis=0, keepdims=True) / _CNT2
    v2 = jnp.sum(tt * tt, axis=0, keepdims=True) / _CNT2 - m2 * m2
    out = _leaky((tt - m2) * (1.0 / jnp.sqrt(v2 + _EPS)))
    out_ref[...] = out.reshape(_B, _N, 8)


def kernel(x, W0, W1, Wc):
    xt = jnp.pad(jnp.transpose(x, (0, 2, 1)), ((0, 0), (0, 0), (0, 5)))
    W0a = W0[:, :3]
    W0b = W0[:, 3:]
    w0at = jnp.pad(W0a.T, ((0, 5), (0, 0)))
    wbat = jnp.pad((W0b - W0a).T, ((0, 5), (0, 0)))
    w1ct = jnp.transpose(W1.reshape(_H, 6, _H), (1, 2, 0))
    wct = jnp.pad(Wc.T, ((0, 0), (0, 5)))

    xj, s0, ss0 = pl.pallas_call(
        _knn_kernel,
        grid=(_B,),
        in_specs=[pl.BlockSpec((1, 3, _N), lambda b: (b, 0, 0)),
                  pl.BlockSpec((1, _N, 8), lambda b: (b, 0, 0)),
                  pl.BlockSpec((8, _H), lambda b: (0, 0)),
                  pl.BlockSpec((8, _H), lambda b: (0, 0))],
        out_specs=[pl.BlockSpec((1, _K, _N, 8), lambda b: (b, 0, 0, 0)),
                   pl.BlockSpec((1, _H), lambda b: (0, 0)),
                   pl.BlockSpec((1, _H), lambda b: (0, 0))],
        out_shape=[jax.ShapeDtypeStruct((_B, _K, _N, 8), jnp.float32),
                   jax.ShapeDtypeStruct((1, _H), jnp.float32),
                   jax.ShapeDtypeStruct((1, _H), jnp.float32)],
    )(x, xt, w0at, wbat)

    x1, s1, ss1 = pl.pallas_call(
        _feat_kernel,
        grid=(_B, _NT),
        in_specs=[pl.BlockSpec((1, _TN, 8), lambda b, i: (b, i, 0)),
                  pl.BlockSpec((1, _K, _TN, 8), lambda b, i: (b, 0, i, 0)),
                  pl.BlockSpec((1, _H), lambda b, i: (0, 0)),
                  pl.BlockSpec((1, _H), lambda b, i: (0, 0)),
                  pl.BlockSpec((8, _H), lambda b, i: (0, 0)),
                  pl.BlockSpec((8, _H), lambda b, i: (0, 0)),
                  pl.BlockSpec((6, _H, _H), lambda b, i: (0, 0, 0))],
        out_specs=[pl.BlockSpec((1, _TN, _H), lambda b, i: (b, i, 0)),
                   pl.BlockSpec((1, _H), lambda b, i: (0, 0)),
                   pl.BlockSpec((1, _H), lambda b, i: (0, 0))],
        out_shape=[jax.ShapeDtypeStruct((_B, _N, _H), jnp.float32),
                   jax.ShapeDtypeStruct((1, _H), jnp.float32),
                   jax.ShapeDtypeStruct((1, _H), jnp.float32)],
    )(xt, xj, s0, ss0, w0at, wbat, w1ct)

    res = pl.pallas_call(
        _head_kernel,
        out_shape=jax.ShapeDtypeStruct((_B, _N, 8), jnp.float32),
    )(x1, s1, ss1, wct)
    return jnp.transpose(res[:, :, :3], (0, 2, 1))
